# baseline (speedup 1.0000x reference)
import functools

import jax
import jax.numpy as jnp
from jax import lax
from jax.experimental import pallas as pl
from jax.experimental.pallas import tpu as pltpu
from jax.experimental.pallas import tpu_sc as plsc

_NC = 2
_NS = 16
_CHUNK = 128
_NB = 4


def _make_sc_segsum(n_pad, n_chunks, dh, with_deg):
    rows_per_sub = n_pad // _NS
    n_groups = n_chunks // _NB
    mesh = plsc.VectorSubcoreMesh(core_axis_name="c", subcore_axis_name="s")
    out_type = [jax.ShapeDtypeStruct((_NC, n_pad, dh), jnp.float32)]
    scratch = [
        pltpu.VMEM((n_chunks, _CHUNK), jnp.int32),
        pltpu.VMEM((n_chunks, _CHUNK), jnp.int32),
    ]
    scratch += [pltpu.VMEM((_CHUNK, dh), jnp.float32)] * _NB
    scratch += [pltpu.VMEM_SHARED((n_pad, dh), jnp.float32)]
    scratch += [pltpu.SemaphoreType.DMA] * (2 * _NB)
    if with_deg:
        out_type.append(jax.ShapeDtypeStruct((_NC, n_pad, 16), jnp.float32))
        scratch += [
            pltpu.VMEM((_CHUNK, 16), jnp.float32),
            pltpu.VMEM_SHARED((n_pad, 16), jnp.float32),
        ]

    def body(vals, srcs, dsts, z_row, z_deg, ones, agg_out, deg_out,
             src_v, dst_v, rows, acc_sh, gsem, ssem,
             ones_v=None, deg_sh=None):
        c = lax.axis_index("c")
        s = lax.axis_index("s")
        sl = pl.ds(s * rows_per_sub, rows_per_sub)
        pltpu.sync_copy(z_row.at[sl], acc_sh.at[sl])
        if with_deg:
            pltpu.sync_copy(z_deg.at[sl], deg_sh.at[sl])
            pltpu.sync_copy(ones, ones_v)
        pltpu.sync_copy(srcs.at[c, s], src_v)
        pltpu.sync_copy(dsts.at[s], dst_v)
        pltpu.async_copy(vals.at[src_v.at[0]], rows[0], gsem[0])
        pltpu.async_copy(vals.at[src_v.at[1]], rows[1], gsem[1])
        plsc.subcore_barrier()

        def group(g, carry):
            i_base = g * _NB
            for k in range(_NB):
                i = i_base + k
                b2 = (k + 2) % _NB
                pltpu.make_async_copy(
                    vals.at[src_v.at[i]], rows[k], gsem[k]).wait()
                pltpu.async_copy(
                    rows[k], acc_sh.at[dst_v.at[i]], ssem[k], add=True)
                if with_deg:
                    pred = (c == 0) if k % 2 == 0 else (c != 0)

                    @pl.when(pred)
                    def _():
                        pltpu.sync_copy(
                            ones_v, deg_sh.at[dst_v.at[i]], add=True)
                if k >= 2:
                    pltpu.make_async_copy(
                        rows[b2], acc_sh.at[dst_v.at[i - 2]],
                        ssem[b2]).wait()

                    @pl.when(g < n_groups - 1)
                    def _():
                        pltpu.async_copy(
                            vals.at[src_v.at[i + 2]], rows[b2], gsem[b2])
                else:
                    @pl.when(g > 0)
                    def _():
                        pltpu.make_async_copy(
                            rows[b2], acc_sh.at[dst_v.at[i - 2]],
                            ssem[b2]).wait()
                    pltpu.async_copy(
                        vals.at[src_v.at[i + 2]], rows[b2], gsem[b2])
            return carry

        lax.fori_loop(0, n_groups, group, 0)

        pltpu.make_async_copy(
            rows[2], acc_sh.at[dst_v.at[n_chunks - 2]], ssem[2]).wait()
        pltpu.make_async_copy(
            rows[3], acc_sh.at[dst_v.at[n_chunks - 1]], ssem[3]).wait()

        plsc.subcore_barrier()
        pltpu.sync_copy(acc_sh.at[sl], agg_out.at[c, sl])
        if with_deg:
            pltpu.sync_copy(deg_sh.at[sl], deg_out.at[c, sl])

    if with_deg:
        def body_wd(vals, srcs, dsts, z_row, z_deg, ones, agg_out, deg_out,
                    src_v, dst_v, r0, r1, r2, r3, acc_sh,
                    g0, g1, g2, g3, s0, s1, s2, s3, ones_v, deg_sh):
            body(vals, srcs, dsts, z_row, z_deg, ones, agg_out, deg_out,
                 src_v, dst_v, [r0, r1, r2, r3], acc_sh,
                 [g0, g1, g2, g3], [s0, s1, s2, s3], ones_v, deg_sh)
        fn = body_wd
    else:
        def body_nd(vals, srcs, dsts, z_row, agg_out,
                    src_v, dst_v, r0, r1, r2, r3, acc_sh,
                    g0, g1, g2, g3, s0, s1, s2, s3):
            body(vals, srcs, dsts, z_row, None, None, agg_out, None,
                 src_v, dst_v, [r0, r1, r2, r3], acc_sh,
                 [g0, g1, g2, g3], [s0, s1, s2, s3])
        fn = body_nd

    return functools.partial(
        pl.kernel, mesh=mesh, out_type=tuple(out_type),
        scratch_types=scratch,
        compiler_params=pltpu.CompilerParams(use_tc_tiling_on_sc=False))(fn)


def _proj2_body(x_ref, wla_ref, wlb_ref, wr_ref, b_ref, p_ref, q_ref):
    x = x_ref[...]
    p_ref[0] = jnp.dot(x, wla_ref[...], preferred_element_type=jnp.float32)
    p_ref[1] = jnp.dot(x, wlb_ref[...], preferred_element_type=jnp.float32)
    q_ref[...] = (jnp.dot(x, wr_ref[...], preferred_element_type=jnp.float32)
                  + b_ref[...])


def _combine_proj2_body(aggp_ref, degp_ref, q_ref, wla_ref, wlb_ref, wr_ref,
                        b_ref, p2_ref, q2_ref):
    n = q_ref.shape[0]
    agg = jnp.concatenate([aggp_ref[0, :n, :], aggp_ref[1, :n, :]], axis=1)
    deg = degp_ref[0, :n, 0:1] + degp_ref[1, :n, 0:1]
    h = jnp.maximum(agg / jnp.maximum(deg, 1.0) + q_ref[...], 0.0)
    p2_ref[0] = jnp.dot(h, wla_ref[...], preferred_element_type=jnp.float32)
    p2_ref[1] = jnp.dot(h, wlb_ref[...], preferred_element_type=jnp.float32)
    q2_ref[...] = (jnp.dot(h, wr_ref[...], preferred_element_type=jnp.float32)
                   + b_ref[...])


def _combine_final_body(aggp_ref, degp_ref, q_ref, w3_ref, b3_ref,
                        h_ref, out_ref):
    n = q_ref.shape[0]
    agg = jnp.concatenate([aggp_ref[0, :n, :], aggp_ref[1, :n, :]], axis=1)
    deg = degp_ref[0, :n, 0:1] + degp_ref[1, :n, 0:1]
    h = jnp.maximum(agg / jnp.maximum(deg, 1.0) + q_ref[...], 0.0)
    h_ref[...] = h
    out_ref[...] = (jnp.dot(h, w3_ref[...], preferred_element_type=jnp.float32)
                    + b3_ref[...])


def kernel(x, edge_index, W_l1, W_r1, b1, W_l2, W_r2, b2, W3, b3):
    n, d = x.shape
    h_dim = W_l1.shape[1]
    dh = h_dim // 2
    o_dim = W3.shape[1]
    e = edge_index.shape[1]

    n_chunks = -(-e // (_NS * _CHUNK * _NB)) * _NB
    e_pad = n_chunks * _CHUNK * _NS
    n_pad = -(-(n + 1) // (_NS * 8)) * (_NS * 8)

    src = jnp.concatenate(
        [edge_index[0], jnp.zeros((e_pad - e,), jnp.int32)])
    dst = jnp.concatenate(
        [edge_index[1], jnp.full((e_pad - e,), n, jnp.int32)])
    src = src.reshape(_NS, n_chunks, _CHUNK)
    src2 = jnp.stack([src, src + n])
    dst = dst.reshape(_NS, n_chunks, _CHUNK)

    z_row = jnp.zeros((n_pad, dh), jnp.float32)
    z_deg = jnp.zeros((n_pad, 16), jnp.float32)
    ones = jnp.ones((_CHUNK, 16), jnp.float32)

    segsum_d = _make_sc_segsum(n_pad, n_chunks, dh, True)
    segsum = _make_sc_segsum(n_pad, n_chunks, dh, False)

    b1r = b1.reshape(1, h_dim)
    b2r = b2.reshape(1, h_dim)
    w3p = jnp.zeros((h_dim, h_dim), jnp.float32).at[:, :o_dim].set(W3)
    b3p = jnp.zeros((1, h_dim), jnp.float32).at[0, :o_dim].set(b3)

    f32 = jnp.float32
    p1, q1 = pl.pallas_call(
        _proj2_body,
        out_shape=(jax.ShapeDtypeStruct((_NC, n, dh), f32),
                   jax.ShapeDtypeStruct((n, h_dim), f32)),
    )(x, W_l1[:, :dh], W_l1[:, dh:], W_r1, b1r)

    agg1p, degp = segsum_d(
        p1.reshape(_NC * n, dh), src2, dst, z_row, z_deg, ones)

    p2, q2 = pl.pallas_call(
        _combine_proj2_body,
        out_shape=(jax.ShapeDtypeStruct((_NC, n, dh), f32),
                   jax.ShapeDtypeStruct((n, h_dim), f32)),
    )(agg1p, degp, q1, W_l2[:, :dh], W_l2[:, dh:], W_r2, b2r)

    (agg2p,) = segsum(p2.reshape(_NC * n, dh), src2, dst, z_row)

    h2, out_p = pl.pallas_call(
        _combine_final_body,
        out_shape=(jax.ShapeDtypeStruct((n, h_dim), f32),
                   jax.ShapeDtypeStruct((n, h_dim), f32)),
    )(agg2p, degp, q2, w3p, b3p)

    return out_p[:, :o_dim], h2


# speedup vs baseline: 1.0599x; 1.0599x over previous
import functools

import jax
import jax.numpy as jnp
from jax import lax
from jax.experimental import pallas as pl
from jax.experimental.pallas import tpu as pltpu
from jax.experimental.pallas import tpu_sc as plsc

_NC = 2
_NS = 16
_CHUNK = 128
_NB = 4


def _make_sc_segsum(n_pad, n_chunks, dh, with_deg):
    rows_per_sub = n_pad // _NS
    n_trips = n_chunks // _NB
    mesh = plsc.VectorSubcoreMesh(core_axis_name="c", subcore_axis_name="s")
    out_type = [jax.ShapeDtypeStruct((_NC, n_pad, dh), jnp.float32)]
    scratch = [
        pltpu.VMEM((n_chunks, _CHUNK), jnp.int32),
        pltpu.VMEM((n_chunks, _CHUNK), jnp.int32),
    ]
    scratch += [pltpu.VMEM((_CHUNK, dh), jnp.float32)] * _NB
    scratch += [pltpu.VMEM_SHARED((n_pad, dh), jnp.float32)]
    scratch += [pltpu.SemaphoreType.DMA] * _NB
    if with_deg:
        out_type.append(jax.ShapeDtypeStruct((_NC, n_pad, 16), jnp.float32))
        scratch += [
            pltpu.VMEM((_CHUNK, 16), jnp.float32),
            pltpu.VMEM_SHARED((n_pad, 16), jnp.float32),
        ]

    def body(vals, srcs, dsts, z_row, z_deg, ones, agg_out, deg_out,
             src_v, dst_v, rows, acc_sh, gsem,
             ones_v=None, deg_sh=None):
        c = lax.axis_index("c")
        s = lax.axis_index("s")
        sl = pl.ds(s * rows_per_sub, rows_per_sub)
        pltpu.sync_copy(z_row.at[sl], acc_sh.at[sl])
        if with_deg:
            pltpu.sync_copy(z_deg.at[sl], deg_sh.at[sl])
            pltpu.sync_copy(ones, ones_v)
        pltpu.sync_copy(srcs.at[c, s], src_v)
        pltpu.sync_copy(dsts.at[s], dst_v)
        pltpu.async_copy(vals.at[src_v.at[0]], rows[0], gsem[0])
        pltpu.async_copy(vals.at[src_v.at[1]], rows[1], gsem[1])
        plsc.subcore_barrier()

        def trip(t, carry):
            j0 = _NB * t
            for k in range(_NB):
                j = j0 + k
                bn = (k + 2) % _NB
                @pl.when(j + 2 < n_chunks)
                def _():
                    pltpu.async_copy(
                        vals.at[src_v.at[j + 2]], rows[bn], gsem[bn])
                pltpu.make_async_copy(
                    vals.at[src_v.at[j]], rows[k], gsem[k]).wait()
                pltpu.sync_copy(rows[k], acc_sh.at[dst_v.at[j]], add=True)
                if with_deg:
                    pred = (c == 0) if k % 2 == 0 else (c != 0)

                    @pl.when(pred)
                    def _():
                        pltpu.sync_copy(
                            ones_v, deg_sh.at[dst_v.at[j]], add=True)
            return carry

        lax.fori_loop(0, n_trips, trip, 0)

        plsc.subcore_barrier()
        pltpu.sync_copy(acc_sh.at[sl], agg_out.at[c, sl])
        if with_deg:
            pltpu.sync_copy(deg_sh.at[sl], deg_out.at[c, sl])

    if with_deg:
        def body_wd(vals, srcs, dsts, z_row, z_deg, ones, agg_out, deg_out,
                    src_v, dst_v, r0, r1, r2, r3, acc_sh,
                    g0, g1, g2, g3, ones_v, deg_sh):
            body(vals, srcs, dsts, z_row, z_deg, ones, agg_out, deg_out,
                 src_v, dst_v, [r0, r1, r2, r3], acc_sh,
                 [g0, g1, g2, g3], ones_v, deg_sh)
        fn = body_wd
    else:
        def body_nd(vals, srcs, dsts, z_row, agg_out,
                    src_v, dst_v, r0, r1, r2, r3, acc_sh,
                    g0, g1, g2, g3):
            body(vals, srcs, dsts, z_row, None, None, agg_out, None,
                 src_v, dst_v, [r0, r1, r2, r3], acc_sh,
                 [g0, g1, g2, g3])
        fn = body_nd

    return functools.partial(
        pl.kernel, mesh=mesh, out_type=tuple(out_type),
        scratch_types=scratch,
        compiler_params=pltpu.CompilerParams(use_tc_tiling_on_sc=False))(fn)


def _proj2_body(x_ref, wla_ref, wlb_ref, wr_ref, b_ref, p_ref, q_ref):
    x = x_ref[...]
    p_ref[0] = jnp.dot(x, wla_ref[...], preferred_element_type=jnp.float32)
    p_ref[1] = jnp.dot(x, wlb_ref[...], preferred_element_type=jnp.float32)
    q_ref[...] = (jnp.dot(x, wr_ref[...], preferred_element_type=jnp.float32)
                  + b_ref[...])


def _combine_proj2_body(aggp_ref, degp_ref, q_ref, wla_ref, wlb_ref, wr_ref,
                        b_ref, p2_ref, q2_ref):
    n = q_ref.shape[0]
    agg = jnp.concatenate([aggp_ref[0, :n, :], aggp_ref[1, :n, :]], axis=1)
    deg = degp_ref[0, :n, 0:1] + degp_ref[1, :n, 0:1]
    h = jnp.maximum(agg / jnp.maximum(deg, 1.0) + q_ref[...], 0.0)
    p2_ref[0] = jnp.dot(h, wla_ref[...], preferred_element_type=jnp.float32)
    p2_ref[1] = jnp.dot(h, wlb_ref[...], preferred_element_type=jnp.float32)
    q2_ref[...] = (jnp.dot(h, wr_ref[...], preferred_element_type=jnp.float32)
                   + b_ref[...])


def _combine_final_body(aggp_ref, degp_ref, q_ref, w3_ref, b3_ref,
                        h_ref, out_ref):
    n = q_ref.shape[0]
    agg = jnp.concatenate([aggp_ref[0, :n, :], aggp_ref[1, :n, :]], axis=1)
    deg = degp_ref[0, :n, 0:1] + degp_ref[1, :n, 0:1]
    h = jnp.maximum(agg / jnp.maximum(deg, 1.0) + q_ref[...], 0.0)
    h_ref[...] = h
    out_ref[...] = (jnp.dot(h, w3_ref[...], preferred_element_type=jnp.float32)
                    + b3_ref[...])


def kernel(x, edge_index, W_l1, W_r1, b1, W_l2, W_r2, b2, W3, b3):
    n, d = x.shape
    h_dim = W_l1.shape[1]
    dh = h_dim // 2
    o_dim = W3.shape[1]
    e = edge_index.shape[1]

    n_chunks = -(-e // (_NS * _CHUNK * _NB)) * _NB
    e_pad = n_chunks * _CHUNK * _NS
    n_pad = -(-(n + 1) // (_NS * 8)) * (_NS * 8)

    src = jnp.concatenate(
        [edge_index[0], jnp.zeros((e_pad - e,), jnp.int32)])
    dst = jnp.concatenate(
        [edge_index[1], jnp.full((e_pad - e,), n, jnp.int32)])
    src = src.reshape(_NS, n_chunks, _CHUNK)
    src2 = jnp.stack([src, src + n])
    dst = dst.reshape(_NS, n_chunks, _CHUNK)

    z_row = jnp.zeros((n_pad, dh), jnp.float32)
    z_deg = jnp.zeros((n_pad, 16), jnp.float32)
    ones = jnp.ones((_CHUNK, 16), jnp.float32)

    segsum_d = _make_sc_segsum(n_pad, n_chunks, dh, True)
    segsum = _make_sc_segsum(n_pad, n_chunks, dh, False)

    b1r = b1.reshape(1, h_dim)
    b2r = b2.reshape(1, h_dim)
    w3p = jnp.zeros((h_dim, h_dim), jnp.float32).at[:, :o_dim].set(W3)
    b3p = jnp.zeros((1, h_dim), jnp.float32).at[0, :o_dim].set(b3)

    f32 = jnp.float32
    p1, q1 = pl.pallas_call(
        _proj2_body,
        out_shape=(jax.ShapeDtypeStruct((_NC, n, dh), f32),
                   jax.ShapeDtypeStruct((n, h_dim), f32)),
    )(x, W_l1[:, :dh], W_l1[:, dh:], W_r1, b1r)

    agg1p, degp = segsum_d(
        p1.reshape(_NC * n, dh), src2, dst, z_row, z_deg, ones)

    p2, q2 = pl.pallas_call(
        _combine_proj2_body,
        out_shape=(jax.ShapeDtypeStruct((_NC, n, dh), f32),
                   jax.ShapeDtypeStruct((n, h_dim), f32)),
    )(agg1p, degp, q1, W_l2[:, :dh], W_l2[:, dh:], W_r2, b2r)

    (agg2p,) = segsum(p2.reshape(_NC * n, dh), src2, dst, z_row)

    h2, out_p = pl.pallas_call(
        _combine_final_body,
        out_shape=(jax.ShapeDtypeStruct((n, h_dim), f32),
                   jax.ShapeDtypeStruct((n, h_dim), f32)),
    )(agg2p, degp, q2, w3p, b3p)

    return out_p[:, :o_dim], h2


# speedup vs baseline: 1.3478x; 1.2716x over previous
import functools

import jax
import jax.numpy as jnp
from jax import lax
from jax.experimental import pallas as pl
from jax.experimental.pallas import tpu as pltpu
from jax.experimental.pallas import tpu_sc as plsc

_NC = 2
_NS = 16
_CHUNK = 128


def _make_sc_segsum(n_pad, n_chunks, dh, with_deg):
    rows_per_sub = n_pad // _NS
    n_pairs = n_chunks // 2
    mesh = plsc.VectorSubcoreMesh(core_axis_name="c", subcore_axis_name="s")
    out_type = [jax.ShapeDtypeStruct((_NC, n_pad, dh), jnp.float32)]
    scratch = [
        pltpu.VMEM((n_chunks, _CHUNK), jnp.int32),
        pltpu.VMEM((n_chunks, _CHUNK), jnp.int32),
        pltpu.VMEM((_CHUNK, dh), jnp.float32),
        pltpu.VMEM((_CHUNK, dh), jnp.float32),
        pltpu.VMEM_SHARED((n_pad, dh), jnp.float32),
        pltpu.SemaphoreType.DMA,
        pltpu.SemaphoreType.DMA,
    ]
    if with_deg:
        out_type.append(jax.ShapeDtypeStruct((_NC, n_pad, 16), jnp.float32))
        scratch += [
            pltpu.VMEM((_CHUNK, 16), jnp.float32),
            pltpu.VMEM_SHARED((n_pad, 16), jnp.float32),
        ]

    def body(vals, srcs, dsts, z_row, z_deg, ones, agg_out, deg_out,
             src_v, dst_v, rows_a, rows_b, acc_sh, sem_a, sem_b,
             ones_v=None, deg_sh=None):
        c = lax.axis_index("c")
        s = lax.axis_index("s")
        sl = pl.ds(s * rows_per_sub, rows_per_sub)
        pltpu.sync_copy(z_row.at[sl], acc_sh.at[sl])
        if with_deg:
            pltpu.sync_copy(z_deg.at[sl], deg_sh.at[sl])
            pltpu.sync_copy(ones, ones_v)
        pltpu.sync_copy(srcs.at[c, s], src_v)
        pltpu.sync_copy(dsts.at[s], dst_v)
        pltpu.async_copy(vals.at[src_v.at[0]], rows_a, sem_a)
        plsc.subcore_barrier()

        def pair(t, carry):
            j0 = 2 * t
            pltpu.async_copy(vals.at[src_v.at[j0 + 1]], rows_b, sem_b)
            pltpu.make_async_copy(vals.at[src_v.at[j0]], rows_a, sem_a).wait()
            pltpu.sync_copy(rows_a, acc_sh.at[dst_v.at[j0]], add=True)
            if with_deg:
                @pl.when(c == 0)
                def _():
                    pltpu.sync_copy(ones_v, deg_sh.at[dst_v.at[j0]], add=True)

            @pl.when(t < n_pairs - 1)
            def _():
                pltpu.async_copy(vals.at[src_v.at[j0 + 2]], rows_a, sem_a)

            pltpu.make_async_copy(
                vals.at[src_v.at[j0 + 1]], rows_b, sem_b).wait()
            pltpu.sync_copy(rows_b, acc_sh.at[dst_v.at[j0 + 1]], add=True)
            if with_deg:
                @pl.when(c != 0)
                def _():
                    pltpu.sync_copy(
                        ones_v, deg_sh.at[dst_v.at[j0 + 1]], add=True)
            return carry

        lax.fori_loop(0, n_pairs, pair, 0)

        plsc.subcore_barrier()
        pltpu.sync_copy(acc_sh.at[sl], agg_out.at[c, sl])
        if with_deg:
            pltpu.sync_copy(deg_sh.at[sl], deg_out.at[c, sl])

    if with_deg:
        def body_wd(vals, srcs, dsts, z_row, z_deg, ones, agg_out, deg_out,
                    src_v, dst_v, rows_a, rows_b, acc_sh, sem_a, sem_b,
                    ones_v, deg_sh):
            body(vals, srcs, dsts, z_row, z_deg, ones, agg_out, deg_out,
                 src_v, dst_v, rows_a, rows_b, acc_sh, sem_a, sem_b,
                 ones_v, deg_sh)
        fn = body_wd
    else:
        def body_nd(vals, srcs, dsts, z_row, agg_out,
                    src_v, dst_v, rows_a, rows_b, acc_sh, sem_a, sem_b):
            body(vals, srcs, dsts, z_row, None, None, agg_out, None,
                 src_v, dst_v, rows_a, rows_b, acc_sh, sem_a, sem_b)
        fn = body_nd

    return functools.partial(
        pl.kernel, mesh=mesh, out_type=tuple(out_type),
        scratch_types=scratch,
        compiler_params=pltpu.CompilerParams(use_tc_tiling_on_sc=False))(fn)


def _proj2_body(x_ref, wla_ref, wlb_ref, wr_ref, b_ref, p_ref, q_ref):
    x = x_ref[...]
    p_ref[0] = jnp.dot(x, wla_ref[...], preferred_element_type=jnp.float32)
    p_ref[1] = jnp.dot(x, wlb_ref[...], preferred_element_type=jnp.float32)
    q_ref[...] = (jnp.dot(x, wr_ref[...], preferred_element_type=jnp.float32)
                  + b_ref[...])


def _combine_proj2_body(aggp_ref, degp_ref, q_ref, wla_ref, wlb_ref, wr_ref,
                        b_ref, p2_ref, q2_ref):
    n = q_ref.shape[0]
    agg = jnp.concatenate([aggp_ref[0, :n, :], aggp_ref[1, :n, :]], axis=1)
    deg = degp_ref[0, :n, 0:1] + degp_ref[1, :n, 0:1]
    h = jnp.maximum(agg / jnp.maximum(deg, 1.0) + q_ref[...], 0.0)
    p2_ref[0] = jnp.dot(h, wla_ref[...], preferred_element_type=jnp.float32)
    p2_ref[1] = jnp.dot(h, wlb_ref[...], preferred_element_type=jnp.float32)
    q2_ref[...] = (jnp.dot(h, wr_ref[...], preferred_element_type=jnp.float32)
                   + b_ref[...])


def _combine_final_body(aggp_ref, degp_ref, q_ref, w3_ref, b3_ref,
                        h_ref, out_ref):
    n = q_ref.shape[0]
    agg = jnp.concatenate([aggp_ref[0, :n, :], aggp_ref[1, :n, :]], axis=1)
    deg = degp_ref[0, :n, 0:1] + degp_ref[1, :n, 0:1]
    h = jnp.maximum(agg / jnp.maximum(deg, 1.0) + q_ref[...], 0.0)
    h_ref[...] = h
    out_ref[...] = (jnp.dot(h, w3_ref[...], preferred_element_type=jnp.float32)
                    + b3_ref[...])


def kernel(x, edge_index, W_l1, W_r1, b1, W_l2, W_r2, b2, W3, b3):
    n, d = x.shape
    h_dim = W_l1.shape[1]
    dh = h_dim // 2
    o_dim = W3.shape[1]
    e = edge_index.shape[1]

    n_chunks = -(-e // (_NS * _CHUNK))
    n_chunks += n_chunks % 2
    e_pad = n_chunks * _CHUNK * _NS
    n_pad = -(-(n + 1) // (_NS * 8)) * (_NS * 8)

    src = jnp.concatenate(
        [edge_index[0], jnp.zeros((e_pad - e,), jnp.int32)])
    dst = jnp.concatenate(
        [edge_index[1], jnp.full((e_pad - e,), n, jnp.int32)])
    src = src.reshape(_NS, n_chunks, _CHUNK)
    src2 = jnp.stack([src, src + n])
    dst = dst.reshape(_NS, n_chunks, _CHUNK)

    z_row = jnp.zeros((n_pad, dh), jnp.float32)
    z_deg = jnp.zeros((n_pad, 16), jnp.float32)
    ones = jnp.ones((_CHUNK, 16), jnp.float32)

    segsum_d = _make_sc_segsum(n_pad, n_chunks, dh, True)
    segsum = _make_sc_segsum(n_pad, n_chunks, dh, False)

    b1r = b1.reshape(1, h_dim)
    b2r = b2.reshape(1, h_dim)
    w3p = jnp.zeros((h_dim, h_dim), jnp.float32).at[:, :o_dim].set(W3)
    b3p = jnp.zeros((1, h_dim), jnp.float32).at[0, :o_dim].set(b3)

    f32 = jnp.float32
    p1, q1 = pl.pallas_call(
        _proj2_body,
        out_shape=(jax.ShapeDtypeStruct((_NC, n, dh), f32),
                   jax.ShapeDtypeStruct((n, h_dim), f32)),
    )(x, W_l1[:, :dh], W_l1[:, dh:], W_r1, b1r)

    agg1p, degp = segsum_d(
        p1.reshape(_NC * n, dh), src2, dst, z_row, z_deg, ones)

    p2, q2 = pl.pallas_call(
        _combine_proj2_body,
        out_shape=(jax.ShapeDtypeStruct((_NC, n, dh), f32),
                   jax.ShapeDtypeStruct((n, h_dim), f32)),
    )(agg1p, degp, q1, W_l2[:, :dh], W_l2[:, dh:], W_r2, b2r)

    (agg2p,) = segsum(p2.reshape(_NC * n, dh), src2, dst, z_row)

    h2, out_p = pl.pallas_call(
        _combine_final_body,
        out_shape=(jax.ShapeDtypeStruct((n, h_dim), f32),
                   jax.ShapeDtypeStruct((n, h_dim), f32)),
    )(agg2p, degp, q2, w3p, b3p)

    return out_p[:, :o_dim], h2


# speedup vs baseline: 1.3759x; 1.0209x over previous
import functools

import jax
import jax.numpy as jnp
from jax import lax
from jax.experimental import pallas as pl
from jax.experimental.pallas import tpu as pltpu
from jax.experimental.pallas import tpu_sc as plsc

_NC = 2
_NS = 16
_CHUNK = 128


def _make_sc_segsum(n_pad, n_chunks, dh, with_deg):
    rows_per_sub = n_pad // _NS
    n_pairs = n_chunks // 2
    mesh = plsc.VectorSubcoreMesh(core_axis_name="c", subcore_axis_name="s")
    out_type = [jax.ShapeDtypeStruct((_NC, n_pad, dh), jnp.float32)]
    scratch = [
        pltpu.VMEM((n_chunks, _CHUNK), jnp.int32),
        pltpu.VMEM((n_chunks, _CHUNK), jnp.int32),
        pltpu.VMEM((_CHUNK, dh), jnp.float32),
        pltpu.VMEM((_CHUNK, dh), jnp.float32),
        pltpu.VMEM_SHARED((n_pad, dh), jnp.float32),
        pltpu.SemaphoreType.DMA,
        pltpu.SemaphoreType.DMA,
    ]
    if with_deg:
        out_type.append(jax.ShapeDtypeStruct((_NC, n_pad, 16), jnp.float32))
        scratch += [
            pltpu.VMEM((_CHUNK, 16), jnp.float32),
            pltpu.VMEM_SHARED((n_pad, 16), jnp.float32),
        ]

    def body(vals, srcs, dsts, z_row, z_deg, ones, agg_out, deg_out,
             src_v, dst_v, rows_a, rows_b, acc_sh, sem_a, sem_b,
             ones_v=None, deg_sh=None):
        c = lax.axis_index("c")
        s = lax.axis_index("s")
        sl = pl.ds(s * rows_per_sub, rows_per_sub)
        pltpu.sync_copy(z_row.at[sl], acc_sh.at[sl])
        if with_deg:
            pltpu.sync_copy(z_deg.at[sl], deg_sh.at[sl])
            pltpu.sync_copy(ones, ones_v)
        pltpu.sync_copy(srcs.at[c, s], src_v)
        pltpu.sync_copy(dsts.at[s], dst_v)
        pltpu.async_copy(vals.at[src_v.at[0]], rows_a, sem_a)
        plsc.subcore_barrier()

        def pair(t, carry):
            j0 = 2 * t
            pltpu.async_copy(vals.at[src_v.at[j0 + 1]], rows_b, sem_b)
            pltpu.make_async_copy(vals.at[src_v.at[j0]], rows_a, sem_a).wait()
            pltpu.sync_copy(rows_a, acc_sh.at[dst_v.at[j0]], add=True)

            @pl.when(t < n_pairs - 1)
            def _():
                pltpu.async_copy(vals.at[src_v.at[j0 + 2]], rows_a, sem_a)

            if with_deg:
                @pl.when(c == 0)
                def _():
                    pltpu.sync_copy(ones_v, deg_sh.at[dst_v.at[j0]], add=True)

            pltpu.make_async_copy(
                vals.at[src_v.at[j0 + 1]], rows_b, sem_b).wait()
            pltpu.sync_copy(rows_b, acc_sh.at[dst_v.at[j0 + 1]], add=True)
            if with_deg:
                @pl.when(c != 0)
                def _():
                    pltpu.sync_copy(
                        ones_v, deg_sh.at[dst_v.at[j0 + 1]], add=True)
            return carry

        lax.fori_loop(0, n_pairs, pair, 0)

        plsc.subcore_barrier()
        pltpu.sync_copy(acc_sh.at[sl], agg_out.at[c, sl])
        if with_deg:
            pltpu.sync_copy(deg_sh.at[sl], deg_out.at[c, sl])

    if with_deg:
        def body_wd(vals, srcs, dsts, z_row, z_deg, ones, agg_out, deg_out,
                    src_v, dst_v, rows_a, rows_b, acc_sh, sem_a, sem_b,
                    ones_v, deg_sh):
            body(vals, srcs, dsts, z_row, z_deg, ones, agg_out, deg_out,
                 src_v, dst_v, rows_a, rows_b, acc_sh, sem_a, sem_b,
                 ones_v, deg_sh)
        fn = body_wd
    else:
        def body_nd(vals, srcs, dsts, z_row, agg_out,
                    src_v, dst_v, rows_a, rows_b, acc_sh, sem_a, sem_b):
            body(vals, srcs, dsts, z_row, None, None, agg_out, None,
                 src_v, dst_v, rows_a, rows_b, acc_sh, sem_a, sem_b)
        fn = body_nd

    return functools.partial(
        pl.kernel, mesh=mesh, out_type=tuple(out_type),
        scratch_types=scratch,
        compiler_params=pltpu.CompilerParams(use_tc_tiling_on_sc=False))(fn)


def _proj2_body(x_ref, wla_ref, wlb_ref, wr_ref, b_ref, p_ref, q_ref):
    x = x_ref[...]
    p_ref[0] = jnp.dot(x, wla_ref[...], preferred_element_type=jnp.float32)
    p_ref[1] = jnp.dot(x, wlb_ref[...], preferred_element_type=jnp.float32)
    q_ref[...] = (jnp.dot(x, wr_ref[...], preferred_element_type=jnp.float32)
                  + b_ref[...])


def _combine_proj2_body(aggp_ref, degp_ref, q_ref, wla_ref, wlb_ref, wr_ref,
                        b_ref, p2_ref, q2_ref):
    n = q_ref.shape[0]
    agg = jnp.concatenate([aggp_ref[0, :n, :], aggp_ref[1, :n, :]], axis=1)
    deg = degp_ref[0, :n, 0:1] + degp_ref[1, :n, 0:1]
    h = jnp.maximum(agg / jnp.maximum(deg, 1.0) + q_ref[...], 0.0)
    p2_ref[0] = jnp.dot(h, wla_ref[...], preferred_element_type=jnp.float32)
    p2_ref[1] = jnp.dot(h, wlb_ref[...], preferred_element_type=jnp.float32)
    q2_ref[...] = (jnp.dot(h, wr_ref[...], preferred_element_type=jnp.float32)
                   + b_ref[...])


def _combine_final_body(aggp_ref, degp_ref, q_ref, w3_ref, b3_ref,
                        h_ref, out_ref):
    n = q_ref.shape[0]
    agg = jnp.concatenate([aggp_ref[0, :n, :], aggp_ref[1, :n, :]], axis=1)
    deg = degp_ref[0, :n, 0:1] + degp_ref[1, :n, 0:1]
    h = jnp.maximum(agg / jnp.maximum(deg, 1.0) + q_ref[...], 0.0)
    h_ref[...] = h
    out_ref[...] = (jnp.dot(h, w3_ref[...], preferred_element_type=jnp.float32)
                    + b3_ref[...])


def kernel(x, edge_index, W_l1, W_r1, b1, W_l2, W_r2, b2, W3, b3):
    n, d = x.shape
    h_dim = W_l1.shape[1]
    dh = h_dim // 2
    o_dim = W3.shape[1]
    e = edge_index.shape[1]

    n_chunks = -(-e // (_NS * _CHUNK))
    n_chunks += n_chunks % 2
    e_pad = n_chunks * _CHUNK * _NS
    n_pad = -(-(n + 1) // (_NS * 8)) * (_NS * 8)

    src = jnp.concatenate(
        [edge_index[0], jnp.zeros((e_pad - e,), jnp.int32)])
    dst = jnp.concatenate(
        [edge_index[1], jnp.full((e_pad - e,), n, jnp.int32)])
    src = src.reshape(_NS, n_chunks, _CHUNK)
    src2 = jnp.stack([src, src + n])
    dst = dst.reshape(_NS, n_chunks, _CHUNK)

    z_row = jnp.zeros((n_pad, dh), jnp.float32)
    z_deg = jnp.zeros((n_pad, 16), jnp.float32)
    ones = jnp.ones((_CHUNK, 16), jnp.float32)

    segsum_d = _make_sc_segsum(n_pad, n_chunks, dh, True)
    segsum = _make_sc_segsum(n_pad, n_chunks, dh, False)

    b1r = b1.reshape(1, h_dim)
    b2r = b2.reshape(1, h_dim)
    w3p = jnp.zeros((h_dim, h_dim), jnp.float32).at[:, :o_dim].set(W3)
    b3p = jnp.zeros((1, h_dim), jnp.float32).at[0, :o_dim].set(b3)

    f32 = jnp.float32
    p1, q1 = pl.pallas_call(
        _proj2_body,
        out_shape=(jax.ShapeDtypeStruct((_NC, n, dh), f32),
                   jax.ShapeDtypeStruct((n, h_dim), f32)),
    )(x, W_l1[:, :dh], W_l1[:, dh:], W_r1, b1r)

    agg1p, degp = segsum_d(
        p1.reshape(_NC * n, dh), src2, dst, z_row, z_deg, ones)

    p2, q2 = pl.pallas_call(
        _combine_proj2_body,
        out_shape=(jax.ShapeDtypeStruct((_NC, n, dh), f32),
                   jax.ShapeDtypeStruct((n, h_dim), f32)),
    )(agg1p, degp, q1, W_l2[:, :dh], W_l2[:, dh:], W_r2, b2r)

    (agg2p,) = segsum(p2.reshape(_NC * n, dh), src2, dst, z_row)

    h2, out_p = pl.pallas_call(
        _combine_final_body,
        out_shape=(jax.ShapeDtypeStruct((n, h_dim), f32),
                   jax.ShapeDtypeStruct((n, h_dim), f32)),
    )(agg2p, degp, q2, w3p, b3p)

    return out_p[:, :o_dim], h2
